# baseline (device time: 25221 ns/iter reference)
import jax
import jax.numpy as jnp
from jax import lax
from jax.experimental import pallas as pl
from jax.experimental.pallas import tpu as pltpu


def kernel(dy, W):
    m, k = dy.shape
    n, _ = W.shape

    def body(dy_ref, w_ref, out_ref, send_buf, recv_buf, send_sem, recv_sem):
        my_x = lax.axis_index("x")
        my_y = lax.axis_index("y")

        send_buf[...] = lax.dot_general(
            dy_ref[...],
            w_ref[...],
            dimension_numbers=(((1,), (1,)), ((), ())),
            preferred_element_type=jnp.float32,
        )

        rdma = pltpu.make_async_remote_copy(
            src_ref=send_buf,
            dst_ref=recv_buf,
            send_sem=send_sem,
            recv_sem=recv_sem,
            device_id=(my_x, 1 - my_y),
            device_id_type=pltpu.DeviceIdType.MESH,
        )
        rdma.start()
        rdma.wait()

        out_ref[...] = send_buf[...] + recv_buf[...]

    return pl.pallas_call(
        body,
        out_shape=jax.ShapeDtypeStruct((m, n), jnp.float32),
        in_specs=[
            pl.BlockSpec(memory_space=pltpu.VMEM),
            pl.BlockSpec(memory_space=pltpu.VMEM),
        ],
        out_specs=pl.BlockSpec(memory_space=pltpu.VMEM),
        scratch_shapes=[
            pltpu.VMEM((m, n), jnp.float32),
            pltpu.VMEM((m, n), jnp.float32),
            pltpu.SemaphoreType.DMA,
            pltpu.SemaphoreType.DMA,
        ],
    )(dy, W)


# device time: 24343 ns/iter; 1.0361x vs baseline; 1.0361x over previous
import jax
import jax.numpy as jnp
from jax import lax
from jax.experimental import pallas as pl
from jax.experimental.pallas import tpu as pltpu

NCHUNKS = 2


def kernel(dy, W):
    m, k = dy.shape
    n = W.shape[0]
    half = m // 2
    rows = half // NCHUNKS

    def body(dy_ref, w_ref, out_ref, sbuf, rbuf, ys_sem, yr_sem, xs_sem, xr_sem):
        my_x = lax.axis_index("x")
        my_y = lax.axis_index("y")
        base = my_x * half

        y_rdmas = []
        for i in range(NCHUNKS):
            r0 = base + i * rows
            sbuf[i, :, :] = lax.dot_general(
                dy_ref[pl.ds(r0, rows), :],
                w_ref[...],
                dimension_numbers=(((1,), (1,)), ((), ())),
                preferred_element_type=jnp.float32,
            )
            rd = pltpu.make_async_remote_copy(
                src_ref=sbuf.at[i],
                dst_ref=rbuf.at[i],
                send_sem=ys_sem.at[i],
                recv_sem=yr_sem.at[i],
                device_id=(my_x, 1 - my_y),
                device_id_type=pltpu.DeviceIdType.MESH,
            )
            rd.start()
            y_rdmas.append(rd)

        x_rdmas = []
        for i in range(NCHUNKS):
            r0 = base + i * rows
            y_rdmas[i].wait()
            out_ref[pl.ds(r0, rows), :] = sbuf[i, :, :] + rbuf[i, :, :]
            rd = pltpu.make_async_remote_copy(
                src_ref=out_ref.at[pl.ds(r0, rows)],
                dst_ref=out_ref.at[pl.ds(r0, rows)],
                send_sem=xs_sem.at[i],
                recv_sem=xr_sem.at[i],
                device_id=(1 - my_x, my_y),
                device_id_type=pltpu.DeviceIdType.MESH,
            )
            rd.start()
            x_rdmas.append(rd)

        for i in range(NCHUNKS):
            x_rdmas[i].wait()

    return pl.pallas_call(
        body,
        out_shape=jax.ShapeDtypeStruct((m, n), jnp.float32),
        in_specs=[
            pl.BlockSpec(memory_space=pltpu.VMEM),
            pl.BlockSpec(memory_space=pltpu.VMEM),
        ],
        out_specs=pl.BlockSpec(memory_space=pltpu.VMEM),
        scratch_shapes=[
            pltpu.VMEM((NCHUNKS, rows, n), jnp.float32),
            pltpu.VMEM((NCHUNKS, rows, n), jnp.float32),
            pltpu.SemaphoreType.DMA((NCHUNKS,)),
            pltpu.SemaphoreType.DMA((NCHUNKS,)),
            pltpu.SemaphoreType.DMA((NCHUNKS,)),
            pltpu.SemaphoreType.DMA((NCHUNKS,)),
        ],
    )(dy, W)
